# Initial kernel scaffold; baseline (speedup 1.0000x reference)
import jax
import jax.numpy as jnp
from jax.experimental import pallas as pl


def kernel(inputs, degs):
    raise NotImplementedError("write your pallas kernel here")


# speedup vs baseline: 1.2329x; 1.2329x over previous
import functools

import jax
import jax.numpy as jnp
from jax import lax
from jax.experimental import pallas as pl
from jax.experimental.pallas import tpu as pltpu, tpu_sc as plsc

BATCH = 16384
NCLASS = 360

_info = plsc.get_sparse_core_info()
_NC, _NS, _L = _info.num_cores, _info.num_subcores, _info.num_lanes
_NW = _NC * _NS
_ROWS_PER_W = BATCH // _NW
_BLK = _L
_NBLK = _ROWS_PER_W // _BLK
_UNROLL = 4


def _tec_body(inputs_hbm, degs_hbm, out_hbm, buf, degs_v, out_v):
    wid = lax.axis_index("s") * _NC + lax.axis_index("c")
    base = wid * _ROWS_PER_W

    pltpu.sync_copy(degs_hbm, degs_v)
    lane_base = lax.iota(jnp.int32, _L) * NCLASS

    def do_block(blk, _):
        row0 = base + blk * _BLK
        pltpu.sync_copy(inputs_hbm.at[pl.ds(row0 * NCLASS, _BLK * NCLASS)], buf)

        def col_step(t, carry):
            m, bi = carry
            j0 = t * _UNROLL
            for u in range(_UNROLL):
                jvec = jnp.full((_L,), j0 + u, dtype=jnp.int32)
                x = plsc.load_gather(buf, [lane_base + jvec])
                gt = x > m
                m = jnp.where(gt, x, m)
                bi = jnp.where(gt, jvec, bi)
            return m, bi

        m0 = jnp.full((_L,), -jnp.inf, dtype=jnp.float32)
        bi0 = jnp.zeros((_L,), dtype=jnp.int32)
        _, bi = lax.fori_loop(0, NCLASS // _UNROLL, col_step, (m0, bi0))

        d = plsc.load_gather(degs_v, [bi])
        out_v[pl.ds(blk * _BLK, _BLK)] = d
        return ()

    lax.fori_loop(0, _NBLK, do_block, ())
    pltpu.sync_copy(out_v, out_hbm.at[pl.ds(base, _ROWS_PER_W)])


@jax.jit
def kernel(inputs, degs):
    mesh = plsc.VectorSubcoreMesh(core_axis_name="c", subcore_axis_name="s")
    run = functools.partial(
        pl.kernel,
        mesh=mesh,
        out_type=jax.ShapeDtypeStruct((BATCH,), jnp.float32),
        compiler_params=pltpu.CompilerParams(
            use_tc_tiling_on_sc=False, needs_layout_passes=False
        ),
        scratch_types=[
            pltpu.VMEM((_BLK * NCLASS,), jnp.float32),
            pltpu.VMEM((NCLASS,), jnp.float32),
            pltpu.VMEM((_ROWS_PER_W,), jnp.float32),
        ],
    )(_tec_body)
    return run(inputs.reshape(-1), degs)


# speedup vs baseline: 1.6172x; 1.3117x over previous
import functools

import jax
import jax.numpy as jnp
from jax import lax
from jax.experimental import pallas as pl
from jax.experimental.pallas import tpu as pltpu, tpu_sc as plsc

BATCH = 16384
NCLASS = 360

_info = plsc.get_sparse_core_info()
_NC, _NS, _L = _info.num_cores, _info.num_subcores, _info.num_lanes
_NW = _NC * _NS
_ROWS_PER_W = BATCH // _NW
_SB_ROWS = 64
_NSB = _ROWS_PER_W // _SB_ROWS
_SUB = _SB_ROWS // _L
_NACC = 4
_CHUNK = NCLASS // _NACC
_SB_WORDS = _SB_ROWS * NCLASS
_PAD = _CHUNK + 8


def _tec_body(inputs_hbm, degs_hbm, out_hbm, buf0, buf1, degs_v, out_v, sem0, sem1):
    wid = lax.axis_index("s") * _NC + lax.axis_index("c")
    base = wid * _ROWS_PER_W

    pltpu.sync_copy(degs_hbm, degs_v)

    lanes = lax.iota(jnp.int32, _L)
    lane_base = [lanes * NCLASS + a * _CHUNK for a in range(_NACC)]

    bufs = (buf0, buf1)
    sems = (sem0, sem1)

    def start_copy(sb):
        k = sb & 1
        return pltpu.async_copy(
            inputs_hbm.at[pl.ds((base + sb * _SB_ROWS) * NCLASS, _SB_WORDS)],
            bufs[k].at[pl.ds(0, _SB_WORDS)],
            sems[k],
        )

    pending = start_copy(0)
    for sb in range(_NSB):
        buf = bufs[sb & 1]
        pending.wait()
        if sb + 1 < _NSB:
            pending = start_copy(sb + 1)

        for b in range(_SUB):
            boff = b * _L * NCLASS

            def col_step(t, carry, _boff=boff, _buf=buf):
                ms, bis, tvec = carry
                window = _buf.at[pl.ds(_boff, _L * NCLASS)]
                new_ms, new_bis = [], []
                for a in range(_NACC):
                    x = plsc.load_gather(window, [lane_base[a] + tvec])
                    gt = x > ms[a]
                    new_ms.append(jnp.where(gt, x, ms[a]))
                    new_bis.append(jnp.where(gt, tvec, bis[a]))
                return tuple(new_ms), tuple(new_bis), tvec + 1

            m0 = tuple(jnp.full((_L,), -jnp.inf, jnp.float32) for _ in range(_NACC))
            b0 = tuple(jnp.zeros((_L,), jnp.int32) for _ in range(_NACC))
            t0 = jnp.zeros((_L,), jnp.int32)
            ms, bis, _ = lax.fori_loop(0, _CHUNK, col_step, (m0, b0, t0))

            m, bi = ms[0], bis[0]
            for a in range(1, _NACC):
                gt = ms[a] > m
                m = jnp.where(gt, ms[a], m)
                bi = jnp.where(gt, bis[a] + a * _CHUNK, bi)

            d = plsc.load_gather(degs_v, [bi])
            out_v[pl.ds(sb * _SB_ROWS + b * _L, _L)] = d

    pltpu.sync_copy(out_v, out_hbm.at[pl.ds(base, _ROWS_PER_W)])


@jax.jit
def kernel(inputs, degs):
    mesh = plsc.VectorSubcoreMesh(core_axis_name="c", subcore_axis_name="s")
    run = functools.partial(
        pl.kernel,
        mesh=mesh,
        out_type=jax.ShapeDtypeStruct((BATCH,), jnp.float32),
        compiler_params=pltpu.CompilerParams(
            use_tc_tiling_on_sc=False, needs_layout_passes=False
        ),
        scratch_types=[
            pltpu.VMEM((_SB_WORDS + _PAD,), jnp.float32),
            pltpu.VMEM((_SB_WORDS + _PAD,), jnp.float32),
            pltpu.VMEM((NCLASS,), jnp.float32),
            pltpu.VMEM((_ROWS_PER_W,), jnp.float32),
            pltpu.SemaphoreType.DMA,
            pltpu.SemaphoreType.DMA,
        ],
    )(_tec_body)
    return run(inputs.reshape(-1), degs)


# speedup vs baseline: 1.6199x; 1.0017x over previous
import functools

import jax
import jax.numpy as jnp
from jax import lax
from jax.experimental import pallas as pl
from jax.experimental.pallas import tpu as pltpu, tpu_sc as plsc

BATCH = 16384
NCLASS = 360

_info = plsc.get_sparse_core_info()
_NC, _NS, _L = _info.num_cores, _info.num_subcores, _info.num_lanes
_NW = _NC * _NS
_ROWS_PER_W = BATCH // _NW
_SB_ROWS = 64
_NSB = _ROWS_PER_W // _SB_ROWS
_SUB = _SB_ROWS // _L
_NACC = 4
_CHUNK = NCLASS // _NACC
_SB_WORDS = _SB_ROWS * NCLASS
_PAD = _CHUNK + 8


def _tec_body(inputs_hbm, degs_hbm, out_hbm, buf0, buf1, degs_v, out_v, sem0, sem1):
    wid = lax.axis_index("s") * _NC + lax.axis_index("c")
    base = wid * _ROWS_PER_W

    pltpu.sync_copy(degs_hbm, degs_v)

    lanes = lax.iota(jnp.int32, _L)

    bufs = (buf0, buf1)
    sems = (sem0, sem1)

    def start_copy(sb):
        k = sb & 1
        return pltpu.async_copy(
            inputs_hbm.at[pl.ds(base + sb * _SB_ROWS, _SB_ROWS), :],
            bufs[k],
            sems[k],
        )

    pending = start_copy(0)
    for sb in range(_NSB):
        buf = bufs[sb & 1]
        pending.wait()
        if sb + 1 < _NSB:
            pending = start_copy(sb + 1)

        for b in range(_SUB):
            rows = lanes + b * _L

            def col_step(t, carry, _rows=rows, _buf=buf):
                ms, bis, tvec = carry
                new_ms, new_bis = [], []
                for a in range(_NACC):
                    x = plsc.load_gather(_buf, [_rows, tvec + a * _CHUNK])
                    gt = x > ms[a]
                    new_ms.append(jnp.where(gt, x, ms[a]))
                    new_bis.append(jnp.where(gt, tvec, bis[a]))
                return tuple(new_ms), tuple(new_bis), tvec + 1

            m0 = tuple(jnp.full((_L,), -jnp.inf, jnp.float32) for _ in range(_NACC))
            b0 = tuple(jnp.zeros((_L,), jnp.int32) for _ in range(_NACC))
            t0 = jnp.zeros((_L,), jnp.int32)
            ms, bis, _ = lax.fori_loop(0, _CHUNK, col_step, (m0, b0, t0))

            m, bi = ms[0], bis[0]
            for a in range(1, _NACC):
                gt = ms[a] > m
                m = jnp.where(gt, ms[a], m)
                bi = jnp.where(gt, bis[a] + a * _CHUNK, bi)

            d = plsc.load_gather(degs_v, [bi])
            out_v[pl.ds(sb * _SB_ROWS + b * _L, _L)] = d

    pltpu.sync_copy(out_v, out_hbm.at[pl.ds(base, _ROWS_PER_W)])


@jax.jit
def kernel(inputs, degs):
    mesh = plsc.VectorSubcoreMesh(core_axis_name="c", subcore_axis_name="s")
    run = functools.partial(
        pl.kernel,
        mesh=mesh,
        out_type=jax.ShapeDtypeStruct((BATCH,), jnp.float32),
        compiler_params=pltpu.CompilerParams(
            use_tc_tiling_on_sc=False, needs_layout_passes=False
        ),
        scratch_types=[
            pltpu.VMEM((_SB_ROWS, NCLASS), jnp.float32),
            pltpu.VMEM((_SB_ROWS, NCLASS), jnp.float32),
            pltpu.VMEM((NCLASS,), jnp.float32),
            pltpu.VMEM((_ROWS_PER_W,), jnp.float32),
            pltpu.SemaphoreType.DMA,
            pltpu.SemaphoreType.DMA,
        ],
    )(_tec_body)
    return run(inputs, degs)
